# baseline (speedup 1.0000x reference)
import functools

import jax
import jax.numpy as jnp
from jax import lax
from jax.experimental import pallas as pl
from jax.experimental.pallas import tpu as pltpu
from jax.experimental.pallas import tpu_sc as plsc

BM = 4096
BE = 1024


def _sub_slices(kk, gw):
    bounds = (0, 2816, 5632, kk)
    pieces = []
    for g in range(kk // gw):
        a = g * gw
        b = a + gw
        for w in range(3):
            lo, hi = bounds[w], bounds[w + 1]
            s, e = max(a, lo), min(b, hi)
            if s < e:
                pieces.append((g, s - a, e - s, w))
    return pieces


def _argmin_body(kk, x_ref, rn_ref, cn_ref, e_ref, idx_ref):
    gw = 2048
    rn = rn_ref[...]
    wv = [None, None, None]
    wi = [None, None, None]
    iota_f = lax.broadcasted_iota(jnp.int32, (1, gw), 1).astype(jnp.float32)
    pieces = _sub_slices(kk, gw)
    for g in range(kk // gw):
        dot = jnp.dot(x_ref[...], e_ref[:, g * gw:(g + 1) * gw],
                      preferred_element_type=jnp.float32)
        d = (rn + dot) + cn_ref[:, g * gw:(g + 1) * gw]
        for (gg, off, width, w) in pieces:
            if gg != g:
                continue
            dsl = d[:, off:off + width]
            dm = jnp.min(dsl, axis=1, keepdims=True)
            cand = jnp.where(dsl == dm, iota_f[:, :width], jnp.float32(3e38))
            imf = jnp.min(cand, axis=1, keepdims=True)
            im = imf.astype(jnp.int32) + (g * gw + off)
            if wv[w] is None:
                wv[w], wi[w] = dm, im
            else:
                upd = dm < wv[w]
                wv[w] = jnp.where(upd, dm, wv[w])
                wi[w] = jnp.where(upd, im, wi[w])
    a0 = wv[0].astype(jnp.bfloat16).astype(jnp.float32)
    upd1 = wv[1] < a0
    a1 = jnp.where(upd1, wv[1], a0).astype(jnp.bfloat16).astype(jnp.float32)
    i01 = jnp.where(upd1, wi[1], wi[0])
    upd2 = wv[2] < a1
    idx_ref[...] = jnp.where(upd2, wi[2], i01)


def _argmin_call(flatten, rownorm, colnorm, embed):
    n, dim = flatten.shape
    kk = embed.shape[1]
    nm = n // BM
    return pl.pallas_call(
        functools.partial(_argmin_body, kk),
        grid=(nm,),
        in_specs=[
            pl.BlockSpec((BM, dim), lambda m: (m, 0)),
            pl.BlockSpec((BM, 1), lambda m: (m, 0)),
            pl.BlockSpec((1, kk), lambda m: (0, 0)),
            pl.BlockSpec((dim, kk), lambda m: (0, 0)),
        ],
        out_specs=pl.BlockSpec((BM, 1), lambda m: (m, 0)),
        out_shape=jax.ShapeDtypeStruct((n, 1), jnp.int32),
    )(flatten, rownorm, colnorm, embed)


def _make_sc_gather(v, d, b):
    info = plsc.get_sparse_core_info()
    nc, ns = info.num_cores, info.num_subcores
    nw = nc * ns
    b_per_w = b // nw
    ch = 128
    nch = b_per_w // ch
    mesh = plsc.VectorSubcoreMesh(core_axis_name="c", subcore_axis_name="s")

    @functools.partial(
        pl.kernel,
        mesh=mesh,
        out_type=jax.ShapeDtypeStruct((b, d), jnp.float32),
        scratch_types=[
            pltpu.VMEM((b_per_w,), jnp.int32),
            pltpu.VMEM((ch, d), jnp.float32),
            pltpu.VMEM((ch, d), jnp.float32),
            pltpu.SemaphoreType.DMA,
            pltpu.SemaphoreType.DMA,
        ],
    )
    def gather_kernel(idx_hbm, table_hbm, out_hbm, idx_v, buf0, buf1, sem0, sem1):
        wid = lax.axis_index("s") * nc + lax.axis_index("c")
        base = wid * b_per_w
        pltpu.sync_copy(idx_hbm.at[pl.ds(base, b_per_w)], idx_v)
        bufs = (buf0, buf1)
        sems = (sem0, sem1)
        copies = [None, None]
        copies[0] = pltpu.async_copy(
            table_hbm.at[idx_v.at[pl.ds(0, ch)]], bufs[0], sems[0])
        for c in range(nch):
            cur = c % 2
            copies[cur].wait()
            if c + 1 < nch:
                nxt = (c + 1) % 2
                copies[nxt] = pltpu.async_copy(
                    table_hbm.at[idx_v.at[pl.ds((c + 1) * ch, ch)]],
                    bufs[nxt], sems[nxt])
            pltpu.sync_copy(bufs[cur], out_hbm.at[pl.ds(base + c * ch, ch)])

    return gather_kernel


def _epilogue_body(nm, inv_n, x_ref, q_ref, out_ref, diff_ref, acc_ref):
    m = pl.program_id(0)
    r = q_ref[...] - x_ref[...]
    out_ref[...] = x_ref[...] + r

    @pl.when(m == 0)
    def _():
        acc_ref[0, 0] = 0.0

    acc_ref[0, 0] += jnp.sum(r * r)

    @pl.when(m == nm - 1)
    def _():
        diff_ref[...] = jnp.full((1, 1), acc_ref[0, 0] * inv_n, jnp.float32)


def _epilogue_call(flatten, quant):
    n, dim = flatten.shape
    nm = n // BE
    return pl.pallas_call(
        functools.partial(_epilogue_body, nm, 1.0 / (n * dim)),
        grid=(nm,),
        in_specs=[
            pl.BlockSpec((BE, dim), lambda m: (m, 0)),
            pl.BlockSpec((BE, dim), lambda m: (m, 0)),
        ],
        out_specs=[
            pl.BlockSpec((BE, dim), lambda m: (m, 0)),
            pl.BlockSpec((1, 1), lambda m: (0, 0)),
        ],
        out_shape=[
            jax.ShapeDtypeStruct((n, dim), jnp.float32),
            jax.ShapeDtypeStruct((1, 1), jnp.float32),
        ],
        scratch_shapes=[pltpu.SMEM((1, 1), jnp.float32)],
    )(flatten, quant)


def kernel(x, embed):
    dim, kk = embed.shape
    flatten = x.reshape(-1, dim)
    n = flatten.shape[0]
    rownorm = jnp.sum(flatten ** 2, axis=1, keepdims=True)
    colnorm = jnp.sum(embed ** 2, axis=0, keepdims=True)

    idx2d = _argmin_call((flatten * -2.0).astype(jnp.bfloat16), rownorm,
                         colnorm, embed.astype(jnp.bfloat16))
    idx = idx2d[:, 0]

    table = embed.T
    quant = _make_sc_gather(kk, dim, n)(idx, table)

    qout, diff2d = _epilogue_call(flatten, quant)

    quantize = qout.reshape(x.shape)
    diff = diff2d[0, 0]
    embed_ind = idx.reshape(x.shape[:-1])
    return (quantize, diff, embed_ind)


# speedup vs baseline: 1.0820x; 1.0820x over previous
import functools

import jax
import jax.numpy as jnp
from jax import lax
from jax.experimental import pallas as pl
from jax.experimental.pallas import tpu as pltpu
from jax.experimental.pallas import tpu_sc as plsc

BM = 2048
BE = 1024


def _sub_slices(kk, gw):
    bounds = (0, 2816, 5632, kk)
    pieces = []
    for g in range(kk // gw):
        a = g * gw
        b = a + gw
        for w in range(3):
            lo, hi = bounds[w], bounds[w + 1]
            s, e = max(a, lo), min(b, hi)
            if s < e:
                pieces.append((g, s - a, e - s, w))
    return pieces


def _argmin_body(kk, x_ref, rn_ref, cn_ref, e_ref, idx_ref):
    gw = 2048
    rn = rn_ref[...]
    wv = [None, None, None]
    wi = [None, None, None]
    iota_f = lax.broadcasted_iota(jnp.int32, (1, gw), 1).astype(jnp.float32)
    pieces = _sub_slices(kk, gw)
    for g in range(kk // gw):
        dot = jnp.dot(x_ref[...], e_ref[:, g * gw:(g + 1) * gw],
                      preferred_element_type=jnp.float32)
        d = (rn + dot) + cn_ref[:, g * gw:(g + 1) * gw]
        for (gg, off, width, w) in pieces:
            if gg != g:
                continue
            dsl = d[:, off:off + width]
            dm = jnp.min(dsl, axis=1, keepdims=True)
            cand = jnp.where(dsl == dm, iota_f[:, :width], jnp.float32(3e38))
            imf = jnp.min(cand, axis=1, keepdims=True)
            im = imf.astype(jnp.int32) + (g * gw + off)
            if wv[w] is None:
                wv[w], wi[w] = dm, im
            else:
                upd = dm < wv[w]
                wv[w] = jnp.where(upd, dm, wv[w])
                wi[w] = jnp.where(upd, im, wi[w])
    a0 = wv[0].astype(jnp.bfloat16).astype(jnp.float32)
    upd1 = wv[1] < a0
    a1 = jnp.where(upd1, wv[1], a0).astype(jnp.bfloat16).astype(jnp.float32)
    i01 = jnp.where(upd1, wi[1], wi[0])
    upd2 = wv[2] < a1
    idx_ref[...] = jnp.where(upd2, wi[2], i01)


def _argmin_call(flatten, rownorm, colnorm, embed):
    n, dim = flatten.shape
    kk = embed.shape[1]
    nm = n // BM
    return pl.pallas_call(
        functools.partial(_argmin_body, kk),
        grid=(nm,),
        in_specs=[
            pl.BlockSpec((BM, dim), lambda m: (m, 0)),
            pl.BlockSpec((BM, 1), lambda m: (m, 0)),
            pl.BlockSpec((1, kk), lambda m: (0, 0)),
            pl.BlockSpec((dim, kk), lambda m: (0, 0)),
        ],
        out_specs=pl.BlockSpec((BM, 1), lambda m: (m, 0)),
        out_shape=jax.ShapeDtypeStruct((n, 1), jnp.int32),
    )(flatten, rownorm, colnorm, embed)


def _make_sc_gather(v, d, b):
    info = plsc.get_sparse_core_info()
    nc, ns = info.num_cores, info.num_subcores
    nw = nc * ns
    b_per_w = b // nw
    ch = 128
    nch = b_per_w // ch
    mesh = plsc.VectorSubcoreMesh(core_axis_name="c", subcore_axis_name="s")

    @functools.partial(
        pl.kernel,
        mesh=mesh,
        out_type=jax.ShapeDtypeStruct((b, d), jnp.float32),
        scratch_types=[
            pltpu.VMEM((b_per_w,), jnp.int32),
            pltpu.VMEM((ch, d), jnp.float32),
            pltpu.VMEM((ch, d), jnp.float32),
            pltpu.SemaphoreType.DMA,
            pltpu.SemaphoreType.DMA,
        ],
    )
    def gather_kernel(idx_hbm, table_hbm, out_hbm, idx_v, buf0, buf1, sem0, sem1):
        wid = lax.axis_index("s") * nc + lax.axis_index("c")
        base = wid * b_per_w
        pltpu.sync_copy(idx_hbm.at[pl.ds(base, b_per_w)], idx_v)
        bufs = (buf0, buf1)
        sems = (sem0, sem1)
        copies = [None, None]
        copies[0] = pltpu.async_copy(
            table_hbm.at[idx_v.at[pl.ds(0, ch)]], bufs[0], sems[0])
        for c in range(nch):
            cur = c % 2
            copies[cur].wait()
            if c + 1 < nch:
                nxt = (c + 1) % 2
                copies[nxt] = pltpu.async_copy(
                    table_hbm.at[idx_v.at[pl.ds((c + 1) * ch, ch)]],
                    bufs[nxt], sems[nxt])
            pltpu.sync_copy(bufs[cur], out_hbm.at[pl.ds(base + c * ch, ch)])

    return gather_kernel


def _epilogue_body(nm, inv_n, x_ref, q_ref, out_ref, diff_ref, acc_ref):
    m = pl.program_id(0)
    r = q_ref[...] - x_ref[...]
    out_ref[...] = x_ref[...] + r

    @pl.when(m == 0)
    def _():
        acc_ref[0, 0] = 0.0

    acc_ref[0, 0] += jnp.sum(r * r)

    @pl.when(m == nm - 1)
    def _():
        diff_ref[...] = jnp.full((1, 1), acc_ref[0, 0] * inv_n, jnp.float32)


def _epilogue_call(flatten, quant):
    n, dim = flatten.shape
    nm = n // BE
    return pl.pallas_call(
        functools.partial(_epilogue_body, nm, 1.0 / (n * dim)),
        grid=(nm,),
        in_specs=[
            pl.BlockSpec((BE, dim), lambda m: (m, 0)),
            pl.BlockSpec((BE, dim), lambda m: (m, 0)),
        ],
        out_specs=[
            pl.BlockSpec((BE, dim), lambda m: (m, 0)),
            pl.BlockSpec((1, 1), lambda m: (0, 0)),
        ],
        out_shape=[
            jax.ShapeDtypeStruct((n, dim), jnp.float32),
            jax.ShapeDtypeStruct((1, 1), jnp.float32),
        ],
        scratch_shapes=[pltpu.SMEM((1, 1), jnp.float32)],
    )(flatten, quant)


def kernel(x, embed):
    dim, kk = embed.shape
    flatten = x.reshape(-1, dim)
    n = flatten.shape[0]
    rownorm = jnp.sum(flatten ** 2, axis=1, keepdims=True)
    colnorm = jnp.sum(embed ** 2, axis=0, keepdims=True)

    idx2d = _argmin_call((flatten * -2.0).astype(jnp.bfloat16), rownorm,
                         colnorm, embed.astype(jnp.bfloat16))
    idx = idx2d[:, 0]

    table = embed.T
    quant = _make_sc_gather(kk, dim, n)(idx, table)

    qout, diff2d = _epilogue_call(flatten, quant)

    quantize = qout.reshape(x.shape)
    diff = diff2d[0, 0]
    embed_ind = idx.reshape(x.shape[:-1])
    return (quantize, diff, embed_ind)
